# Initial kernel scaffold; baseline (speedup 1.0000x reference)
import jax
import jax.numpy as jnp
from jax.experimental import pallas as pl


def kernel(p0, p1, p2, p3, p4, f0, f1, f2, f3, f4, Wc1, gc, bc, Wc2, s3w0, s3g0, s3b0, s3w1, s3g1, s3b1, s2w0, s2g0, s2b0, s2w1, s2g1, s2b1, s1w0, s1g0, s1b0, s1w1, s1g1, s1b1, s0w0, s0g0, s0b0, s0w1, s0g1, s0b1, cls_label):
    raise NotImplementedError("write your pallas kernel here")


# speedup vs baseline: 17.8981x; 17.8981x over previous
import functools
import math

import jax
import jax.numpy as jnp
from jax.experimental import pallas as pl

_F32 = jnp.float32
_BN_EPS = 1e-5
_D_EPS = 1e-8
_BIG = 3.0e38


def _scale_shift(sum_ref, sq_ref, g_ref, b_ref, m):
    inv_m = jnp.float32(1.0 / m)
    mean = sum_ref[...] * inv_m
    var = sq_ref[...] * inv_m - mean * mean
    sc = g_ref[...] * jax.lax.rsqrt(var + _BN_EPS)
    sh = b_ref[...] - mean * sc
    return sc, sh


def _accum_stats(y, osum_ref, osq_ref):
    first = jnp.logical_and(pl.program_id(0) == 0, pl.program_id(1) == 0)

    @pl.when(first)
    def _():
        osum_ref[...] = jnp.zeros_like(osum_ref)
        osq_ref[...] = jnp.zeros_like(osq_ref)

    osum_ref[...] += jnp.sum(y, axis=1, keepdims=True)
    osq_ref[...] += jnp.sum(y * y, axis=1, keepdims=True)


def _mm(a, b):
    return jax.lax.dot_general(a, b, (((1,), (0,)), ((), ())),
                               preferred_element_type=_F32)


def _mm_t(a, b):
    return jax.lax.dot_general(a, b, (((1,), (1,)), ((), ())),
                               preferred_element_type=_F32)


def _three_nn_weights(p1, p2t, t, n2):
    d2 = jnp.zeros((t, n2), dtype=_F32)
    for k in range(3):
        diff = p1[:, k:k + 1] - p2t[k:k + 1, :]
        d2 = d2 + diff * diff
    lane = jax.lax.broadcasted_iota(jnp.int32, (t, n2), 1)
    dd = d2
    dvals, idxs = [], []
    for _ in range(3):
        mk = jnp.min(dd, axis=1, keepdims=True)
        ik = jnp.min(jnp.where(dd == mk, lane, n2), axis=1, keepdims=True)
        dd = jnp.where(lane == ik, _BIG, dd)
        dvals.append(mk)
        idxs.append(ik)
    r = [1.0 / (d + _D_EPS) for d in dvals]
    rtot = r[0] + r[1] + r[2]
    st = jnp.zeros((t, n2), dtype=_F32)
    for k in range(3):
        wk = r[k] / rtot
        st = st + jnp.where(lane == idxs[k], wk, jnp.float32(0.0))
    return st


def _interp_conv_kernel(p1_ref, p2t_ref, f2_ref, fsum_ref, fsq_ref, fg_ref,
                        fb_ref, f1_ref, w0a_ref, w0b_ref,
                        y_ref, osum_ref, osq_ref, *maybe_itp,
                        m_prev, t, n2, save_interp):
    sc, sh = _scale_shift(fsum_ref, fsq_ref, fg_ref, fb_ref, m_prev)
    f2 = f2_ref[0] * sc + sh
    st = _three_nn_weights(p1_ref[0], p2t_ref[0], t, n2)
    interp = _mm_t(f2, st)
    y = _mm(w0a_ref[...], f1_ref[0]) + _mm(w0b_ref[...], interp)
    y_ref[0] = y
    if save_interp:
        maybe_itp[0][0] = interp
    _accum_stats(y, osum_ref, osq_ref)


def _conv_kernel(y_ref, fsum_ref, fsq_ref, fg_ref, fb_ref, w_ref,
                 o_ref, osum_ref, osq_ref, *, m_prev):
    sc, sh = _scale_shift(fsum_ref, fsq_ref, fg_ref, fb_ref, m_prev)
    y = _mm(w_ref[...], y_ref[0] * sc + sh)
    o_ref[0] = y
    _accum_stats(y, osum_ref, osq_ref)


def _s0_conv_kernel(y_ref, fsum_ref, fsq_ref, fg_ref, fb_ref, c_ref, itp_ref,
                    w0a_ref, w0b_ref, o_ref, osum_ref, osq_ref, *, m_prev):
    sc, sh = _scale_shift(fsum_ref, fsq_ref, fg_ref, fb_ref, m_prev)
    x1 = y_ref[0] * sc + sh + c_ref[0]
    y = _mm(w0a_ref[...], x1) + _mm(w0b_ref[...], itp_ref[0])
    o_ref[0] = y
    _accum_stats(y, osum_ref, osq_ref)


def _norm_kernel(y_ref, fsum_ref, fsq_ref, fg_ref, fb_ref, o_ref, *, m_prev):
    sc, sh = _scale_shift(fsum_ref, fsq_ref, fg_ref, fb_ref, m_prev)
    o_ref[0] = y_ref[0] * sc + sh


def _cls_kernel(lbl_ref, wc1_ref, gc_ref, bc_ref, wc2_ref, out_ref, *, b):
    lbl = lbl_ref[...]
    oh = (jax.lax.broadcasted_iota(jnp.int32, (b, 16), 1) == lbl).astype(_F32)
    y = _mm_t(oh, wc1_ref[...])
    mean = jnp.mean(y, axis=0, keepdims=True)
    var = jnp.mean(y * y, axis=0, keepdims=True) - mean * mean
    x = (y - mean) * jax.lax.rsqrt(var + _BN_EPS)
    x = x * gc_ref[...] + bc_ref[...]
    g = 0.5 * x * (1.0 + jax.lax.erf(x * jnp.float32(1.0 / math.sqrt(2.0))))
    out_ref[...] = _mm_t(g, wc2_ref[...])


def _stat_spec(shape):
    return pl.BlockSpec(shape, lambda bb, tt: (0, 0))


def _feat_spec(c, t):
    return pl.BlockSpec((1, c, t), lambda bb, tt: (bb, 0, tt))


def _full_feat_spec(c, n):
    return pl.BlockSpec((1, c, n), lambda bb, tt: (bb, 0, 0))


def _interp_conv(p1, p2t, f2y, fstats, f1, w0a, w0b, *, m_prev, n_tiles,
                 save_interp):
    b, n1, _ = p1.shape
    _, c2, n2 = f2y.shape
    c1 = f1.shape[1]
    o = w0a.shape[0]
    t = n1 // n_tiles
    fsum, fsq, fg, fb = fstats
    out_shape = [jax.ShapeDtypeStruct((b, o, n1), _F32),
                 jax.ShapeDtypeStruct((o, 1), _F32),
                 jax.ShapeDtypeStruct((o, 1), _F32)]
    out_specs = [_feat_spec(o, t), _stat_spec((o, 1)), _stat_spec((o, 1))]
    if save_interp:
        out_shape.append(jax.ShapeDtypeStruct((b, c2, n1), _F32))
        out_specs.append(_feat_spec(c2, t))
    return pl.pallas_call(
        functools.partial(_interp_conv_kernel, m_prev=m_prev, t=t, n2=n2,
                          save_interp=save_interp),
        grid=(b, n_tiles),
        in_specs=[pl.BlockSpec((1, t, 3), lambda bb, tt: (bb, tt, 0)),
                  pl.BlockSpec((1, 3, n2), lambda bb, tt: (bb, 0, 0)),
                  _full_feat_spec(c2, n2),
                  _stat_spec((c2, 1)), _stat_spec((c2, 1)),
                  _stat_spec((c2, 1)), _stat_spec((c2, 1)),
                  _feat_spec(c1, t),
                  _stat_spec(w0a.shape), _stat_spec(w0b.shape)],
        out_specs=out_specs,
        out_shape=out_shape,
    )(p1, p2t, f2y, fsum, fsq, fg, fb, f1, w0a, w0b)


def _conv(y, fstats, w, *, m_prev):
    b, c, n = y.shape
    o = w.shape[0]
    fsum, fsq, fg, fb = fstats
    return pl.pallas_call(
        functools.partial(_conv_kernel, m_prev=m_prev),
        grid=(b, 1),
        in_specs=[_feat_spec(c, n),
                  _stat_spec((c, 1)), _stat_spec((c, 1)),
                  _stat_spec((c, 1)), _stat_spec((c, 1)),
                  _stat_spec(w.shape)],
        out_specs=[_feat_spec(o, n), _stat_spec((o, 1)), _stat_spec((o, 1))],
        out_shape=[jax.ShapeDtypeStruct((b, o, n), _F32),
                   jax.ShapeDtypeStruct((o, 1), _F32),
                   jax.ShapeDtypeStruct((o, 1), _F32)],
    )(y, fsum, fsq, fg, fb, w)


def _s0_conv(y, fstats, c3, itp, w0a, w0b, *, m_prev):
    b, c, n = y.shape
    o = w0a.shape[0]
    c2 = itp.shape[1]
    fsum, fsq, fg, fb = fstats
    return pl.pallas_call(
        functools.partial(_s0_conv_kernel, m_prev=m_prev),
        grid=(b, 1),
        in_specs=[_feat_spec(c, n),
                  _stat_spec((c, 1)), _stat_spec((c, 1)),
                  _stat_spec((c, 1)), _stat_spec((c, 1)),
                  pl.BlockSpec((1, c, 1), lambda bb, tt: (bb, 0, 0)),
                  _feat_spec(c2, n),
                  _stat_spec(w0a.shape), _stat_spec(w0b.shape)],
        out_specs=[_feat_spec(o, n), _stat_spec((o, 1)), _stat_spec((o, 1))],
        out_shape=[jax.ShapeDtypeStruct((b, o, n), _F32),
                   jax.ShapeDtypeStruct((o, 1), _F32),
                   jax.ShapeDtypeStruct((o, 1), _F32)],
    )(y, fsum, fsq, fg, fb, c3, itp, w0a, w0b)


def _final_norm(y, fstats, *, m_prev):
    b, c, n = y.shape
    fsum, fsq, fg, fb = fstats
    return pl.pallas_call(
        functools.partial(_norm_kernel, m_prev=m_prev),
        grid=(b, 1),
        in_specs=[_feat_spec(c, n),
                  _stat_spec((c, 1)), _stat_spec((c, 1)),
                  _stat_spec((c, 1)), _stat_spec((c, 1))],
        out_specs=_feat_spec(c, n),
        out_shape=jax.ShapeDtypeStruct((b, c, n), _F32),
    )(y, fsum, fsq, fg, fb)


def kernel(p0, p1, p2, p3, p4, f0, f1, f2, f3, f4, Wc1, gc, bc, Wc2,
           s3w0, s3g0, s3b0, s3w1, s3g1, s3b1,
           s2w0, s2g0, s2b0, s2w1, s2g1, s2b1,
           s1w0, s1g0, s1b0, s1w1, s1g1, s1b1,
           s0w0, s0g0, s0b0, s0w1, s0g1, s0b1, cls_label):
    b = p0.shape[0]
    n1, n2, n3, n4 = p1.shape[1], p2.shape[1], p3.shape[1], p4.shape[1]

    col = lambda v: v.reshape(-1, 1)
    row = lambda v: v.reshape(1, -1)

    c_col = pl.pallas_call(
        functools.partial(_cls_kernel, b=b),
        out_shape=jax.ShapeDtypeStruct((b, 128), _F32),
    )(cls_label, Wc1, row(gc), row(bc), Wc2)
    c3 = c_col.reshape(b, 128, 1)

    def raw_stats(c, m):
        zero = jnp.zeros((c, 1), _F32)
        one = jnp.ones((c, 1), _F32)
        sq = jnp.full((c, 1), m * (1.0 - _BN_EPS), _F32)
        return (zero, sq, one, zero)

    t = lambda p: jnp.swapaxes(p, 1, 2)

    y30, s30a, s30b = _interp_conv(
        p3, t(p4), f4, raw_stats(f4.shape[1], b * n4), f3,
        s3w0[:, :f3.shape[1]], s3w0[:, f3.shape[1]:],
        m_prev=b * n4, n_tiles=1, save_interp=False)
    st30 = (s30a, s30b, col(s3g0), col(s3b0))
    y31, s31a, s31b = _conv(y30, st30, s3w1, m_prev=b * n3)
    st31 = (s31a, s31b, col(s3g1), col(s3b1))

    y20, s20a, s20b = _interp_conv(
        p2, t(p3), y31, st31, f2,
        s2w0[:, :f2.shape[1]], s2w0[:, f2.shape[1]:],
        m_prev=b * n3, n_tiles=1, save_interp=False)
    st20 = (s20a, s20b, col(s2g0), col(s2b0))
    y21, s21a, s21b = _conv(y20, st20, s2w1, m_prev=b * n2)
    st21 = (s21a, s21b, col(s2g1), col(s2b1))

    y10, s10a, s10b, itp1 = _interp_conv(
        p1, t(p2), y21, st21, f1,
        s1w0[:, :f1.shape[1]], s1w0[:, f1.shape[1]:],
        m_prev=b * n2, n_tiles=4, save_interp=True)
    st10 = (s10a, s10b, col(s1g0), col(s1b0))
    y11, s11a, s11b = _conv(y10, st10, s1w1, m_prev=b * n1)
    st11 = (s11a, s11b, col(s1g1), col(s1b1))

    y00, s00a, s00b = _s0_conv(
        y11, st11, c3, itp1,
        s0w0[:, :128], s0w0[:, 128:], m_prev=b * n1)
    st00 = (s00a, s00b, col(s0g0), col(s0b0))
    y01, s01a, s01b = _conv(y00, st00, s0w1, m_prev=b * n1)
    st01 = (s01a, s01b, col(s0g1), col(s0b1))

    return _final_norm(y01, st01, m_prev=b * n1)


# speedup vs baseline: 22.6227x; 1.2640x over previous
import functools
import math

import jax
import jax.numpy as jnp
from jax.experimental import pallas as pl
from jax.experimental.pallas import tpu as pltpu

_F32 = jnp.float32
_BN_EPS = 1e-5
_D_EPS = 1e-8


def _scale_shift(s, q, g, b, m):
    inv_m = jnp.float32(1.0 / m)
    mean = s * inv_m
    var = q * inv_m - mean * mean
    sc = g * jax.lax.rsqrt(var + _BN_EPS)
    sh = b - mean * sc
    return sc, sh


def _mm(a, b):
    return jax.lax.dot_general(a, b, (((1,), (0,)), ((), ())),
                               preferred_element_type=_F32)


def _rsum(y):
    return jnp.sum(y, axis=1, keepdims=True)


def _three_nn_weights(p1t, p2, t, n2):
    d0 = p2[:, 0:1] - p1t[0:1, :]
    d1 = p2[:, 1:2] - p1t[1:2, :]
    d2c = p2[:, 2:3] - p1t[2:3, :]
    d2 = d0 * d0 + d1 * d1 + d2c * d2c
    sub = jax.lax.broadcasted_iota(jnp.int32, (n2, t), 0)
    key = (jax.lax.bitcast_convert_type(d2, jnp.int32) & (~1023)) | sub
    iks, dks = [], []
    for _ in range(3):
        mk = jnp.min(key, axis=0, keepdims=True)
        key = jnp.where(key == mk, jnp.int32(0x7FFFFFFF), key)
        ik = mk & 1023
        iks.append(ik)
        dks.append(jax.lax.bitcast_convert_type(mk - ik, _F32))
    r = [1.0 / (d + _D_EPS) for d in dks]
    rtot = r[0] + r[1] + r[2]
    s = jnp.zeros((n2, t), dtype=_F32)
    for k in range(3):
        wk = r[k] / rtot
        s = s + jnp.where(sub == iks[k], wk, jnp.float32(0.0))
    return s


def _interp_level(bsz, n1, n2, tile, p1t_ref, p2_ref, f2n_of_b, f1_ref,
                  w0a, w0b, y_out, itp_out=None):
    o = w0a.shape[0]
    acc_s = jnp.zeros((o, 1), _F32)
    acc_q = jnp.zeros((o, 1), _F32)
    nt = n1 // tile
    for b in range(bsz):
        f2n = f2n_of_b(b)
        p2 = p2_ref[b]

        def body(tt, carry, b=b, f2n=f2n, p2=p2):
            a_s, a_q = carry
            sl = pl.ds(tt * tile, tile)
            s = _three_nn_weights(p1t_ref[b, :, sl], p2, tile, n2)
            itp = _mm(f2n, s)
            if itp_out is not None:
                itp_out[b, :, sl] = itp
            y = _mm(w0a, f1_ref[b, :, sl]) + _mm(w0b, itp)
            y_out[b, :, sl] = y
            return (a_s + _rsum(y), a_q + _rsum(y * y))

        acc_s, acc_q = jax.lax.fori_loop(0, nt, body, (acc_s, acc_q))
    return acc_s, acc_q


def _conv_pass(bsz, n, tile, w, src_ref, sc, sh, dst_ref, extra_of_b=None,
               w2=None, src2_ref=None):
    acc_s = jnp.zeros((w.shape[0], 1), _F32)
    acc_q = jnp.zeros((w.shape[0], 1), _F32)
    for b in range(bsz):
        e = extra_of_b(b) if extra_of_b is not None else None

        def body(tt, carry, b=b, e=e):
            a_s, a_q = carry
            sl = pl.ds(tt * tile, tile)
            x = src_ref[b, :, sl] * sc + sh
            if e is not None:
                x = x + e
            y = _mm(w, x)
            if w2 is not None:
                y = y + _mm(w2, src2_ref[b, :, sl])
            dst_ref[b, :, sl] = y
            return (a_s + _rsum(y), a_q + _rsum(y * y))

        acc_s, acc_q = jax.lax.fori_loop(0, n // tile, body, (acc_s, acc_q))
    return acc_s, acc_q


def _kernel_a(p3t_ref, p4_ref, p2t_ref, p3_ref, f4_ref, f3_ref, f2_ref,
              w30a_ref, w30b_ref, w31_ref, g30_ref, b30_ref,
              w20a_ref, w20b_ref, g31_ref, b31_ref,
              w21_ref, g20_ref, b20_ref,
              y21_ref, s21_ref, q21_ref, y30_ref, y31_ref, y20_ref,
              *, bsz, n4, n3, n2):
    s30, q30 = _interp_level(
        bsz, n3, n4, n3, p3t_ref, p4_ref,
        lambda b: f4_ref[b], f3_ref,
        w30a_ref[...], w30b_ref[...], y30_ref)
    sc, sh = _scale_shift(s30, q30, g30_ref[...], b30_ref[...], bsz * n3)
    s31, q31 = _conv_pass(bsz, n3, n3, w31_ref[...], y30_ref, sc, sh, y31_ref)
    sc31, sh31 = _scale_shift(s31, q31, g31_ref[...], b31_ref[...], bsz * n3)

    s20, q20 = _interp_level(
        bsz, n2, n3, n2, p2t_ref, p3_ref,
        lambda b: y31_ref[b] * sc31 + sh31, f2_ref,
        w20a_ref[...], w20b_ref[...], y20_ref)
    sc20, sh20 = _scale_shift(s20, q20, g20_ref[...], b20_ref[...], bsz * n2)
    s21, q21 = _conv_pass(bsz, n2, n2, w21_ref[...], y20_ref, sc20, sh20,
                          y21_ref)
    s21_ref[...] = s21
    q21_ref[...] = q21


def _kernel_b(p1t_ref, p2_ref, f1_ref, y21_ref, s21_ref, q21_ref,
              g21_ref, b21_ref, lbl_ref, wc1_ref, gc_ref, bc_ref, wc2_ref,
              w10a_ref, w10b_ref, w11_ref, g10_ref, b10_ref,
              w00a_ref, w00b_ref, w01_ref, g11_ref, b11_ref,
              g00_ref, b00_ref, g01_ref, b01_ref,
              out_ref, itp_ref, ya_ref, yb_ref, *, bsz, n2, n1, tile):
    lbl = lbl_ref[...]
    oh = (jax.lax.broadcasted_iota(jnp.int32, (16, bsz), 0) == lbl).astype(_F32)
    yc = _mm(wc1_ref[...], oh)
    mean = jnp.mean(yc, axis=1, keepdims=True)
    var = jnp.mean(yc * yc, axis=1, keepdims=True) - mean * mean
    xc = (yc - mean) * jax.lax.rsqrt(var + _BN_EPS)
    xc = xc * gc_ref[...] + bc_ref[...]
    gl = 0.5 * xc * (1.0 + jax.lax.erf(xc * jnp.float32(1.0 / math.sqrt(2.0))))
    ct = _mm(wc2_ref[...], gl)

    sc21, sh21 = _scale_shift(s21_ref[...], q21_ref[...],
                              g21_ref[...], b21_ref[...], bsz * n2)

    s10, q10 = _interp_level(
        bsz, n1, n2, tile, p1t_ref, p2_ref,
        lambda b: y21_ref[b] * sc21 + sh21, f1_ref,
        w10a_ref[...], w10b_ref[...], ya_ref, itp_out=itp_ref)
    sc10, sh10 = _scale_shift(s10, q10, g10_ref[...], b10_ref[...], bsz * n1)
    s11, q11 = _conv_pass(bsz, n1, 2048, w11_ref[...], ya_ref, sc10, sh10,
                          yb_ref)
    sc11, sh11 = _scale_shift(s11, q11, g11_ref[...], b11_ref[...], bsz * n1)

    s00, q00 = _conv_pass(bsz, n1, 2048, w00a_ref[...], yb_ref, sc11, sh11,
                          ya_ref, extra_of_b=lambda b: ct[:, b:b + 1],
                          w2=w00b_ref[...], src2_ref=itp_ref)
    sc00, sh00 = _scale_shift(s00, q00, g00_ref[...], b00_ref[...], bsz * n1)
    s01, q01 = _conv_pass(bsz, n1, 2048, w01_ref[...], ya_ref, sc00, sh00,
                          yb_ref)
    sc01, sh01 = _scale_shift(s01, q01, g01_ref[...], b01_ref[...], bsz * n1)
    for b in range(bsz):

        def body(tt, carry, b=b):
            sl = pl.ds(tt * 2048, 2048)
            out_ref[b, :, sl] = yb_ref[b, :, sl] * sc01 + sh01
            return carry

        jax.lax.fori_loop(0, n1 // 2048, body, 0)


def kernel(p0, p1, p2, p3, p4, f0, f1, f2, f3, f4, Wc1, gc, bc, Wc2,
           s3w0, s3g0, s3b0, s3w1, s3g1, s3b1,
           s2w0, s2g0, s2b0, s2w1, s2g1, s2b1,
           s1w0, s1g0, s1b0, s1w1, s1g1, s1b1,
           s0w0, s0g0, s0b0, s0w1, s0g1, s0b1, cls_label):
    bsz = p0.shape[0]
    n1, n2, n3, n4 = p1.shape[1], p2.shape[1], p3.shape[1], p4.shape[1]
    c3, c2, c1 = f3.shape[1], f2.shape[1], f1.shape[1]

    col = lambda v: v.reshape(-1, 1)
    tr = lambda p: jnp.swapaxes(p, 1, 2)

    vmem3 = lambda c, n: pltpu.VMEM((bsz, c, n), _F32)
    y21, s21, q21 = pl.pallas_call(
        functools.partial(_kernel_a, bsz=bsz, n4=n4, n3=n3, n2=n2),
        out_shape=[jax.ShapeDtypeStruct((bsz, 128, n2), _F32),
                   jax.ShapeDtypeStruct((128, 1), _F32),
                   jax.ShapeDtypeStruct((128, 1), _F32)],
        scratch_shapes=[vmem3(256, n3), vmem3(256, n3), vmem3(128, n2)],
    )(tr(p3), p4, tr(p2), p3, f4, f3, f2,
      s3w0[:, :c3], s3w0[:, c3:], s3w1, col(s3g0), col(s3b0),
      s2w0[:, :c2], s2w0[:, c2:], col(s3g1), col(s3b1),
      s2w1, col(s2g0), col(s2b0))

    return pl.pallas_call(
        functools.partial(_kernel_b, bsz=bsz, n2=n2, n1=n1, tile=512),
        out_shape=jax.ShapeDtypeStruct((bsz, 128, n1), _F32),
        scratch_shapes=[vmem3(128, n1), vmem3(128, n1), vmem3(128, n1)],
    )(tr(p1), p2, f1, y21, s21, q21, col(s2g1), col(s2b1),
      cls_label.reshape(1, -1), Wc1, col(gc), col(bc), Wc2,
      s1w0[:, :c1], s1w0[:, c1:], s1w1, col(s1g0), col(s1b0),
      s0w0[:, :128], s0w0[:, 128:], s0w1, col(s1g1), col(s1b1),
      col(s0g0), col(s0b0), col(s0g1), col(s0b1))


# speedup vs baseline: 23.7661x; 1.0505x over previous
import functools
import math

import jax
import jax.numpy as jnp
from jax.experimental import pallas as pl
from jax.experimental.pallas import tpu as pltpu

_F32 = jnp.float32
_BN_EPS = 1e-5
_D_EPS = 1e-8


def _scale_shift(s, q, g, b, m):
    inv_m = jnp.float32(1.0 / m)
    mean = s * inv_m
    var = q * inv_m - mean * mean
    sc = g * jax.lax.rsqrt(var + _BN_EPS)
    sh = b - mean * sc
    return sc, sh


def _mm(a, b):
    return jax.lax.dot_general(a, b, (((1,), (0,)), ((), ())),
                               preferred_element_type=_F32)


def _rsum(y):
    return jnp.sum(y, axis=1, keepdims=True)


def _fold(w, sc, sh):
    wf = w * jnp.transpose(sc)
    bias = _mm(w, sh)
    return wf, bias


def _three_nn_weights(p1t, p2, t, n2):
    d0 = p2[:, 0:1] - p1t[0:1, :]
    d1 = p2[:, 1:2] - p1t[1:2, :]
    d2c = p2[:, 2:3] - p1t[2:3, :]
    d2 = d0 * d0 + d1 * d1 + d2c * d2c
    sub = jax.lax.broadcasted_iota(jnp.int32, (n2, t), 0)
    key = (jax.lax.bitcast_convert_type(d2, jnp.int32) & (~1023)) | sub
    iks, dks = [], []
    for _ in range(3):
        mk = jnp.min(key, axis=0, keepdims=True)
        key = jnp.where(key == mk, jnp.int32(0x7FFFFFFF), key)
        ik = mk & 1023
        iks.append(ik)
        dks.append(jax.lax.bitcast_convert_type(mk - ik, _F32))
    r = [1.0 / (d + _D_EPS) for d in dks]
    rtot = r[0] + r[1] + r[2]
    s = jnp.zeros((n2, t), dtype=_F32)
    for k in range(3):
        wk = r[k] / rtot
        s = s + jnp.where(sub == iks[k], wk, jnp.float32(0.0))
    return s


def _interp_level(bsz, n1, n2, tile, p1t_ref, p2_ref, z_of_b, f1_ref,
                  w0a, w0b, bias, y_out, itp_out=None):
    o = w0a.shape[0]
    acc_s = jnp.zeros((o, 1), _F32)
    acc_q = jnp.zeros((o, 1), _F32)
    nt = n1 // tile
    unroll = 2 if nt % 2 == 0 else 1

    for b in range(bsz):
        z = z_of_b(b)
        p2 = p2_ref[b]

        def tile_work(sl, z=z, p2=p2, b=b):
            s = _three_nn_weights(p1t_ref[b, :, sl], p2, tile, n2)
            itp = _mm(z, s)
            if itp_out is not None:
                itp_out[b, :, sl] = itp
            y = _mm(w0a, f1_ref[b, :, sl]) + _mm(w0b, itp) + bias
            y_out[b, :, sl] = y
            return _rsum(y), _rsum(y * y)

        if nt == 1:
            ds_, dq = tile_work(slice(0, tile))
            acc_s += ds_
            acc_q += dq
        else:
            def body(tt, carry):
                a_s, a_q = carry
                for u in range(unroll):
                    sl = pl.ds((tt * unroll + u) * tile, tile)
                    ds_, dq = tile_work(sl)
                    a_s += ds_
                    a_q += dq
                return (a_s, a_q)

            acc_s, acc_q = jax.lax.fori_loop(0, nt // unroll, body,
                                             (acc_s, acc_q))
    return acc_s, acc_q


def _conv_pass(bsz, n, tile, wf, bias_of_b, src_ref, dst_ref,
               w2=None, src2_ref=None):
    acc_s = jnp.zeros((wf.shape[0], 1), _F32)
    acc_q = jnp.zeros((wf.shape[0], 1), _F32)
    for b in range(bsz):
        bias = bias_of_b(b)

        def tile_work(sl, bias=bias, b=b):
            y = _mm(wf, src_ref[b, :, sl]) + bias
            if w2 is not None:
                y = y + _mm(w2, src2_ref[b, :, sl])
            dst_ref[b, :, sl] = y
            return _rsum(y), _rsum(y * y)

        if n == tile:
            ds_, dq = tile_work(slice(0, tile))
            acc_s += ds_
            acc_q += dq
        else:
            def body(tt, carry):
                a_s, a_q = carry
                ds_, dq = tile_work(pl.ds(tt * tile, tile))
                return (a_s + ds_, a_q + dq)

            acc_s, acc_q = jax.lax.fori_loop(0, n // tile, body,
                                             (acc_s, acc_q))
    return acc_s, acc_q


def _kernel_a(p3t_ref, p4_ref, p2t_ref, p3_ref, f4_ref, f3_ref, f2_ref,
              w30a_ref, w30b_ref, w31_ref, g30_ref, b30_ref,
              w20a_ref, w20b_ref, g31_ref, b31_ref,
              w21_ref, g20_ref, b20_ref,
              y21_ref, s21_ref, q21_ref, y30_ref, y31_ref, y20_ref,
              *, bsz, n4, n3, n2):
    zero_bias = jnp.zeros((w30a_ref.shape[0], 1), _F32)
    s30, q30 = _interp_level(
        bsz, n3, n4, n3, p3t_ref, p4_ref,
        lambda b: f4_ref[b], f3_ref,
        w30a_ref[...], w30b_ref[...], zero_bias, y30_ref)
    sc, sh = _scale_shift(s30, q30, g30_ref[...], b30_ref[...], bsz * n3)
    wf, bias = _fold(w31_ref[...], sc, sh)
    s31, q31 = _conv_pass(bsz, n3, n3, wf, lambda b: bias, y30_ref, y31_ref)
    sc31, sh31 = _scale_shift(s31, q31, g31_ref[...], b31_ref[...], bsz * n3)

    bias20 = _mm(w20b_ref[...], sh31)
    s20, q20 = _interp_level(
        bsz, n2, n3, n2, p2t_ref, p3_ref,
        lambda b: y31_ref[b] * sc31, f2_ref,
        w20a_ref[...], w20b_ref[...], bias20, y20_ref)
    sc20, sh20 = _scale_shift(s20, q20, g20_ref[...], b20_ref[...], bsz * n2)
    wf21, bias21 = _fold(w21_ref[...], sc20, sh20)
    s21, q21 = _conv_pass(bsz, n2, n2, wf21, lambda b: bias21, y20_ref,
                          y21_ref)
    s21_ref[...] = s21
    q21_ref[...] = q21


def _kernel_b(p1t_ref, p2_ref, f1_ref, y21_ref, s21_ref, q21_ref,
              g21_ref, b21_ref, lbl_ref, wc1_ref, gc_ref, bc_ref, wc2_ref,
              w10a_ref, w10b_ref, w11_ref, g10_ref, b10_ref,
              w00a_ref, w00b_ref, w01_ref, g11_ref, b11_ref,
              g00_ref, b00_ref, g01_ref, b01_ref,
              out_ref, itp_ref, ya_ref, yb_ref, *, bsz, n2, n1, tile):
    lbl = lbl_ref[...]
    oh = (jax.lax.broadcasted_iota(jnp.int32, (16, bsz), 0) == lbl).astype(_F32)
    yc = _mm(wc1_ref[...], oh)
    mean = jnp.mean(yc, axis=1, keepdims=True)
    var = jnp.mean(yc * yc, axis=1, keepdims=True) - mean * mean
    xc = (yc - mean) * jax.lax.rsqrt(var + _BN_EPS)
    xc = xc * gc_ref[...] + bc_ref[...]
    gl = 0.5 * xc * (1.0 + jax.lax.erf(xc * jnp.float32(1.0 / math.sqrt(2.0))))
    ct = _mm(wc2_ref[...], gl)

    sc21, sh21 = _scale_shift(s21_ref[...], q21_ref[...],
                              g21_ref[...], b21_ref[...], bsz * n2)

    bias10 = _mm(w10b_ref[...], sh21)
    s10, q10 = _interp_level(
        bsz, n1, n2, tile, p1t_ref, p2_ref,
        lambda b: y21_ref[b] * sc21, f1_ref,
        w10a_ref[...], w10b_ref[...], bias10, ya_ref, itp_out=itp_ref)
    sc10, sh10 = _scale_shift(s10, q10, g10_ref[...], b10_ref[...], bsz * n1)
    wf11, bias11 = _fold(w11_ref[...], sc10, sh10)
    s11, q11 = _conv_pass(bsz, n1, 2048, wf11, lambda b: bias11, ya_ref,
                          yb_ref)
    sc11, sh11 = _scale_shift(s11, q11, g11_ref[...], b11_ref[...], bsz * n1)

    wf00, bias00c = _fold(w00a_ref[...], sc11, sh11)
    bias00b = _mm(w00b_ref[...], sh21)
    bias00 = bias00c + bias00b
    s00, q00 = _conv_pass(
        bsz, n1, 2048, wf00,
        lambda b: bias00 + _mm(w00a_ref[...], ct[:, b:b + 1]),
        yb_ref, ya_ref, w2=w00b_ref[...], src2_ref=itp_ref)
    sc00, sh00 = _scale_shift(s00, q00, g00_ref[...], b00_ref[...], bsz * n1)
    wf01, bias01 = _fold(w01_ref[...], sc00, sh00)
    s01, q01 = _conv_pass(bsz, n1, 2048, wf01, lambda b: bias01, ya_ref,
                          yb_ref)
    sc01, sh01 = _scale_shift(s01, q01, g01_ref[...], b01_ref[...], bsz * n1)
    for b in range(bsz):

        def body(tt, carry, b=b):
            sl = pl.ds(tt * 2048, 2048)
            out_ref[b, :, sl] = yb_ref[b, :, sl] * sc01 + sh01
            return carry

        jax.lax.fori_loop(0, n1 // 2048, body, 0)


def kernel(p0, p1, p2, p3, p4, f0, f1, f2, f3, f4, Wc1, gc, bc, Wc2,
           s3w0, s3g0, s3b0, s3w1, s3g1, s3b1,
           s2w0, s2g0, s2b0, s2w1, s2g1, s2b1,
           s1w0, s1g0, s1b0, s1w1, s1g1, s1b1,
           s0w0, s0g0, s0b0, s0w1, s0g1, s0b1, cls_label):
    bsz = p0.shape[0]
    n1, n2, n3, n4 = p1.shape[1], p2.shape[1], p3.shape[1], p4.shape[1]
    c3, c2, c1 = f3.shape[1], f2.shape[1], f1.shape[1]

    col = lambda v: v.reshape(-1, 1)
    tr = lambda p: jnp.swapaxes(p, 1, 2)

    vmem3 = lambda c, n: pltpu.VMEM((bsz, c, n), _F32)
    y21, s21, q21 = pl.pallas_call(
        functools.partial(_kernel_a, bsz=bsz, n4=n4, n3=n3, n2=n2),
        out_shape=[jax.ShapeDtypeStruct((bsz, 128, n2), _F32),
                   jax.ShapeDtypeStruct((128, 1), _F32),
                   jax.ShapeDtypeStruct((128, 1), _F32)],
        scratch_shapes=[vmem3(256, n3), vmem3(256, n3), vmem3(128, n2)],
    )(tr(p3), p4, tr(p2), p3, f4, f3, f2,
      s3w0[:, :c3], s3w0[:, c3:], s3w1, col(s3g0), col(s3b0),
      s2w0[:, :c2], s2w0[:, c2:], col(s3g1), col(s3b1),
      s2w1, col(s2g0), col(s2b0))

    return pl.pallas_call(
        functools.partial(_kernel_b, bsz=bsz, n2=n2, n1=n1, tile=512),
        out_shape=jax.ShapeDtypeStruct((bsz, 128, n1), _F32),
        scratch_shapes=[vmem3(128, n1), vmem3(128, n1), vmem3(128, n1)],
    )(tr(p1), p2, f1, y21, s21, q21, col(s2g1), col(s2b1),
      cls_label.reshape(1, -1), Wc1, col(gc), col(bc), Wc2,
      s1w0[:, :c1], s1w0[:, c1:], s1w1, col(s1g0), col(s1b0),
      s0w0[:, :128], s0w0[:, 128:], s0w1, col(s1g1), col(s1b1),
      col(s0g0), col(s0b0), col(s0g1), col(s0b1))
